# baseline (speedup 1.0000x reference)
import functools

import jax
import jax.numpy as jnp
from jax import lax
from jax.experimental import pallas as pl
from jax.experimental.pallas import tpu as pltpu
from jax.experimental.pallas import tpu_sc as plsc

_REGS = 1e-05
_NC, _NS = 2, 16
_NW = _NC * _NS
_D = 16
_NEG = -1e30


def _iota16():
    return lax.iota(jnp.int32, 16)


def _splat_i32(x):
    return jnp.broadcast_to(jnp.asarray(x, jnp.int32), (16,))


def _make_sc_stats(B, L):
    R = B // _NW
    G = (L + 15) // 16
    LP = G * 16
    LA = 96
    LB = L - LA
    UC = 128
    C = 64
    assert B % _NW == 0 and R % UC == 0 and LB <= 128 and R % C == 0

    mesh = plsc.VectorSubcoreMesh(
        core_axis_name="c", subcore_axis_name="s",
        num_cores=_NC, num_subcores=_NS)

    @functools.partial(
        pl.kernel,
        out_type=jax.ShapeDtypeStruct((B, _D), jnp.float32),
        mesh=mesh,
        compiler_params=pltpu.CompilerParams(use_tc_tiling_on_sc=False,
                                             needs_layout_passes=False),
        scratch_types=[
            pltpu.VMEM((R // UC, UC), jnp.int32),
            pltpu.VMEM((R, _D), jnp.float32),
            pltpu.VMEM((C, L), jnp.int32),
            pltpu.VMEM((C, L), jnp.float32),
            pltpu.VMEM((C, LP), jnp.float32),
            pltpu.VMEM((LA, _D), jnp.float32),
            pltpu.VMEM((LB, _D), jnp.float32),
            pltpu.VMEM((LA, _D), jnp.float32),
            pltpu.VMEM((LB, _D), jnp.float32),
            pltpu.VMEM((LA, _D), jnp.float32),
            pltpu.VMEM((LB, _D), jnp.float32),
            pltpu.VMEM((LA, _D), jnp.float32),
            pltpu.VMEM((LB, _D), jnp.float32),
            pltpu.VMEM((R, _D), jnp.float32),
            pltpu.SemaphoreType.DMA,
            pltpu.SemaphoreType.DMA,
            pltpu.SemaphoreType.DMA,
            pltpu.SemaphoreType.DMA,
            pltpu.SemaphoreType.DMA,
        ],
    )
    def sc_stats(user_hbm, items_hbm, reward_hbm, gumbel_hbm, uemb_hbm,
                 iemb_hbm, out_hbm, uidx_v, urows_v, idx_v, rwd_v, gmb_v,
                 ea0_v, eb0_v, ea1_v, eb1_v, ea2_v, eb2_v, ea3_v, eb3_v,
                 out_v, semi, sem0, sem1, sem2, sem3):
        wid = lax.axis_index("s") * _NC + lax.axis_index("c")
        base = wid * R
        iota = _iota16()
        zeros = jnp.zeros((16,), jnp.float32)

        for k in range(R // UC):
            pltpu.sync_copy(user_hbm.at[pl.ds(base + k * UC, UC)],
                            uidx_v.at[k])
        cps = [pltpu.async_copy(uemb_hbm.at[uidx_v.at[k]],
                                urows_v.at[pl.ds(k * UC, UC)], semi)
               for k in range(R // UC)]
        for c in cps:
            c.wait()

        def issue(c, ea, eb, sem):
            pltpu.async_copy(iemb_hbm.at[idx_v.at[c, pl.ds(0, LA)]], ea, sem)
            pltpu.async_copy(iemb_hbm.at[idx_v.at[c, pl.ds(LA, LB)]], eb,
                             sem)

        def wait_set(ea, eb, sem):
            pltpu.make_async_copy(iemb_hbm.at[pl.ds(0, LA)], ea, sem).wait()
            pltpu.make_async_copy(iemb_hbm.at[pl.ds(0, LB)], eb, sem).wait()

        def compute_row(r, c, ea, eb):
            u = urows_v[r, :]
            dcols = [(iota + _splat_i32(k)) & _splat_i32(15)
                     for k in range(_D)]
            u_b = [jnp.take_along_axis(u, dcols[k], axis=0)
                   for k in range(_D)]

            sq = [zeros] * 4
            se_v = zeros
            best = jnp.full((16,), _NEG, jnp.float32)
            bidx = _splat_i32(0)
            blg = jnp.full((16,), _NEG, jnp.float32)
            c_s = _splat_i32(0) + c
            for j in range(G):
                rows = _splat_i32(j * 16) + iota
                full = (j + 1) * 16 <= L
                in_a = j * 16 < LA
                eref = ea if in_a else eb
                if full:
                    rows_c = rows if in_a else rows - LA
                else:
                    valid = rows < L
                    rows_c = jnp.minimum(rows, L - 1) - LA
                acc = [zeros, zeros]
                for d in range(_D):
                    col = plsc.load_gather(eref, [rows_c, dcols[d]])
                    acc[d & 1] = col * u_b[d] + acc[d & 1]
                    if full:
                        sq[d & 3] = col * col + sq[d & 3]
                    else:
                        sq[d & 3] = jnp.where(valid, col * col + sq[d & 3],
                                              sq[d & 3])
                lg = acc[0] + acc[1]
                if not full:
                    lg = jnp.where(valid, lg, _NEG)
                se_v = se_v + jnp.exp(lg)
                nz = lg + gmb_v[c, pl.ds(j * 16, 16)]
                upd = nz > best
                best = jnp.where(upd, nz, best)
                bidx = jnp.where(upd, _splat_i32(j * 16) + iota, bidx)
                blg = jnp.where(upd, lg, blg)
            se = jnp.sum(se_v)

            mx = jnp.max(best)
            midx = jnp.where(best == mx, bidx, _splat_i32(2 ** 30))
            s_id = jnp.min(midx)
            win = midx == _splat_i32(0) + s_id
            sl = jnp.max(jnp.where(win, blg, _NEG))
            rw_g = plsc.load_gather(rwd_v, [c_s, _splat_i32(0) + s_id])
            rw = jnp.max(rw_g)

            u2 = jnp.sum(u * u)
            i2 = jnp.sum(sq[0] + sq[1] + (sq[2] + sq[3]))

            st = jnp.where(iota == 0, sl, zeros)
            st = jnp.where(iota == 2, se, st)
            st = jnp.where(iota == 3, rw, st)
            st = jnp.where(iota == 4, u2, st)
            st = jnp.where(iota == 5, i2, st)
            plsc.store_scatter(out_v, [_splat_i32(0) + r, iota], st)

        def chunk_body(ci, carry):
            cbase = ci * C
            rb = pl.multiple_of(base + cbase, 8)
            c1 = pltpu.async_copy(items_hbm.at[pl.ds(rb, C)], idx_v, semi)
            c2 = pltpu.async_copy(reward_hbm.at[pl.ds(rb, C)], rwd_v, semi)
            c3 = pltpu.async_copy(gumbel_hbm.at[pl.ds(rb, C)],
                                  gmb_v.at[:, pl.ds(0, L)], semi)
            c1.wait()
            c2.wait()
            c3.wait()
            sets = [(ea0_v, eb0_v, sem0), (ea1_v, eb1_v, sem1),
                    (ea2_v, eb2_v, sem2), (ea3_v, eb3_v, sem3)]
            for k, (ea, eb, sem) in enumerate(sets):
                issue(k, ea, eb, sem)

            def quad_body(p, carry2):
                c0 = 4 * p
                for k, (ea, eb, sem) in enumerate(sets):
                    wait_set(ea, eb, sem)
                    compute_row(cbase + c0 + k, c0 + k, ea, eb)

                    @pl.when(c0 + k + 4 < C)
                    def _(k=k, ea=ea, eb=eb, sem=sem):
                        issue(c0 + k + 4, ea, eb, sem)

                return carry2

            lax.fori_loop(0, C // 4, quad_body, 0)
            return carry

        lax.fori_loop(0, R // C, chunk_body, 0)
        pltpu.sync_copy(out_v, out_hbm.at[pl.ds(pl.multiple_of(base, 8), R)])

    return sc_stats


def _tc_reduce(stats, B):
    def body(s_ref, gan_ref, reg_ref):
        s = s_ref[...]
        sl = s[:, 0]
        se = s[:, 2]
        rw = s[:, 3]
        lp = sl - jnp.log(se)
        gan_ref[...] = jnp.reshape(-jnp.mean(lp * rw), (1, 1))
        reg_ref[...] = jnp.reshape(
            _REGS * 0.5 * (jnp.sum(s[:, 4]) + jnp.sum(s[:, 5])), (1, 1))

    gan, reg = pl.pallas_call(
        body,
        out_shape=(jax.ShapeDtypeStruct((1, 1), jnp.float32),
                   jax.ShapeDtypeStruct((1, 1), jnp.float32)),
    )(stats)
    return gan[0, 0], reg[0, 0]


def kernel(user, items, reward, user_embedding, item_embedding):
    B, L = items.shape
    gumbel = jax.random.gumbel(jax.random.key(42), (B, L), jnp.float32)
    sc = _make_sc_stats(B, L)
    stats = sc(user, items, reward, gumbel, user_embedding, item_embedding)
    return _tc_reduce(stats, B)


# speedup vs baseline: 1.2618x; 1.2618x over previous
import functools

import jax
import jax.numpy as jnp
from jax import lax
from jax.experimental import pallas as pl
from jax.experimental.pallas import tpu as pltpu
from jax.experimental.pallas import tpu_sc as plsc

_REGS = 1e-05
_NC, _NS = 2, 16
_NW = _NC * _NS
_D = 16
_NEG = -1e30


def _iota16():
    return lax.iota(jnp.int32, 16)


def _splat_i32(x):
    return jnp.broadcast_to(jnp.asarray(x, jnp.int32), (16,))


def _make_sc_stats(B, L):
    R = B // _NW
    G = (L + 15) // 16
    LP = G * 16
    LA = 96
    LB = L - LA
    UC = 128
    C = 64
    assert B % _NW == 0 and R % UC == 0 and LB <= 128 and R % C == 0

    mesh = plsc.VectorSubcoreMesh(
        core_axis_name="c", subcore_axis_name="s",
        num_cores=_NC, num_subcores=_NS)

    @functools.partial(
        pl.kernel,
        out_type=jax.ShapeDtypeStruct((B, _D), jnp.float32),
        mesh=mesh,
        compiler_params=pltpu.CompilerParams(use_tc_tiling_on_sc=False,
                                             needs_layout_passes=False),
        scratch_types=[
            pltpu.VMEM((R // UC, UC), jnp.int32),
            pltpu.VMEM((R, _D), jnp.float32),
            pltpu.VMEM((C, L), jnp.int32),
            pltpu.VMEM((C, L), jnp.float32),
            pltpu.VMEM((C, LP), jnp.float32),
            pltpu.VMEM((LA, _D), jnp.float32),
            pltpu.VMEM((LB, _D), jnp.float32),
            pltpu.VMEM((LA, _D), jnp.float32),
            pltpu.VMEM((LB, _D), jnp.float32),
            pltpu.VMEM((R, _D), jnp.float32),
            pltpu.SemaphoreType.DMA,
            pltpu.SemaphoreType.DMA,
            pltpu.SemaphoreType.DMA,
        ],
    )
    def sc_stats(user_hbm, items_hbm, reward_hbm, gumbel_hbm, uemb_hbm,
                 iemb_hbm, out_hbm, uidx_v, urows_v, idx_v, rwd_v, gmb_v,
                 ea0_v, eb0_v, ea1_v, eb1_v, out_v,
                 semi, sem0, sem1):
        wid = lax.axis_index("s") * _NC + lax.axis_index("c")
        base = wid * R
        iota = _iota16()
        zeros = jnp.zeros((16,), jnp.float32)

        for k in range(R // UC):
            pltpu.sync_copy(user_hbm.at[pl.ds(base + k * UC, UC)],
                            uidx_v.at[k])
        cps = [pltpu.async_copy(uemb_hbm.at[uidx_v.at[k]],
                                urows_v.at[pl.ds(k * UC, UC)], semi)
               for k in range(R // UC)]
        for c in cps:
            c.wait()

        def issue(c, ea, eb, sem):
            pltpu.async_copy(iemb_hbm.at[idx_v.at[c, pl.ds(0, LA)]], ea, sem)
            pltpu.async_copy(iemb_hbm.at[idx_v.at[c, pl.ds(LA, LB)]], eb,
                             sem)

        def wait_set(ea, eb, sem):
            pltpu.make_async_copy(iemb_hbm.at[pl.ds(0, LA)], ea, sem).wait()
            pltpu.make_async_copy(iemb_hbm.at[pl.ds(0, LB)], eb, sem).wait()

        def compute_row(r, c, ea, eb):
            u = urows_v[r, :]
            dcols = [(iota + _splat_i32(k)) & _splat_i32(15)
                     for k in range(_D)]
            u_b = [jnp.take_along_axis(u, dcols[k], axis=0)
                   for k in range(_D)]

            sq = [zeros] * 4
            se_v = zeros
            best = jnp.full((16,), _NEG, jnp.float32)
            bidx = _splat_i32(0)
            blg = jnp.full((16,), _NEG, jnp.float32)
            c_s = _splat_i32(0) + c
            for j in range(G):
                rows = _splat_i32(j * 16) + iota
                full = (j + 1) * 16 <= L
                in_a = j * 16 < LA
                eref = ea if in_a else eb
                if full:
                    rows_c = rows if in_a else rows - LA
                else:
                    valid = rows < L
                    rows_c = jnp.minimum(rows, L - 1) - LA
                acc = [zeros, zeros]
                for d in range(_D):
                    col = plsc.load_gather(eref, [rows_c, dcols[d]])
                    acc[d & 1] = col * u_b[d] + acc[d & 1]
                    if full:
                        sq[d & 3] = col * col + sq[d & 3]
                    else:
                        sq[d & 3] = jnp.where(valid, col * col + sq[d & 3],
                                              sq[d & 3])
                lg = acc[0] + acc[1]
                if not full:
                    lg = jnp.where(valid, lg, _NEG)
                se_v = se_v + jnp.exp(lg)
                nz = lg + gmb_v[c, pl.ds(j * 16, 16)]
                upd = nz > best
                best = jnp.where(upd, nz, best)
                bidx = jnp.where(upd, _splat_i32(j * 16) + iota, bidx)
                blg = jnp.where(upd, lg, blg)
            se = jnp.sum(se_v)

            mx = jnp.max(best)
            midx = jnp.where(best == mx, bidx, _splat_i32(2 ** 30))
            s_id = jnp.min(midx)
            win = midx == _splat_i32(0) + s_id
            sl = jnp.max(jnp.where(win, blg, _NEG))
            rw_g = plsc.load_gather(rwd_v, [c_s, _splat_i32(0) + s_id])
            rw = jnp.max(rw_g)

            u2 = jnp.sum(u * u)
            i2 = jnp.sum(sq[0] + sq[1] + (sq[2] + sq[3]))

            st = jnp.where(iota == 0, sl, zeros)
            st = jnp.where(iota == 2, se, st)
            st = jnp.where(iota == 3, rw, st)
            st = jnp.where(iota == 4, u2, st)
            st = jnp.where(iota == 5, i2, st)
            plsc.store_scatter(out_v, [_splat_i32(0) + r, iota], st)

        def chunk_body(ci, carry):
            cbase = ci * C
            rb = pl.multiple_of(base + cbase, 8)
            c1 = pltpu.async_copy(items_hbm.at[pl.ds(rb, C)], idx_v, semi)
            c2 = pltpu.async_copy(reward_hbm.at[pl.ds(rb, C)], rwd_v, semi)
            c3 = pltpu.async_copy(gumbel_hbm.at[pl.ds(rb, C)],
                                  gmb_v.at[:, pl.ds(0, L)], semi)
            c1.wait()
            c2.wait()
            c3.wait()
            issue(0, ea0_v, eb0_v, sem0)
            issue(1, ea1_v, eb1_v, sem1)

            def pair_body(p, carry2):
                c0 = 2 * p
                wait_set(ea0_v, eb0_v, sem0)
                compute_row(cbase + c0, c0, ea0_v, eb0_v)

                @pl.when(c0 + 2 < C)
                def _():
                    issue(c0 + 2, ea0_v, eb0_v, sem0)

                wait_set(ea1_v, eb1_v, sem1)
                compute_row(cbase + c0 + 1, c0 + 1, ea1_v, eb1_v)

                @pl.when(c0 + 3 < C)
                def _():
                    issue(c0 + 3, ea1_v, eb1_v, sem1)

                return carry2

            lax.fori_loop(0, C // 2, pair_body, 0)
            return carry

        lax.fori_loop(0, R // C, chunk_body, 0)
        pltpu.sync_copy(out_v, out_hbm.at[pl.ds(pl.multiple_of(base, 8), R)])

    return sc_stats


def _tc_reduce(stats, B):
    def body(s_ref, gan_ref, reg_ref):
        s = s_ref[...]
        sl = s[:, 0]
        se = s[:, 2]
        rw = s[:, 3]
        lp = sl - jnp.log(se)
        gan_ref[...] = jnp.reshape(-jnp.mean(lp * rw), (1, 1))
        reg_ref[...] = jnp.reshape(
            _REGS * 0.5 * (jnp.sum(s[:, 4]) + jnp.sum(s[:, 5])), (1, 1))

    gan, reg = pl.pallas_call(
        body,
        out_shape=(jax.ShapeDtypeStruct((1, 1), jnp.float32),
                   jax.ShapeDtypeStruct((1, 1), jnp.float32)),
    )(stats)
    return gan[0, 0], reg[0, 0]


def kernel(user, items, reward, user_embedding, item_embedding):
    B, L = items.shape
    gumbel = jax.random.gumbel(jax.random.key(42), (B, L), jnp.float32)
    sc = _make_sc_stats(B, L)
    stats = sc(user, items, reward, gumbel, user_embedding, item_embedding)
    return _tc_reduce(stats, B)


# speedup vs baseline: 1.2878x; 1.0206x over previous
import functools

import jax
import jax.numpy as jnp
from jax import lax
from jax.experimental import pallas as pl
from jax.experimental.pallas import tpu as pltpu
from jax.experimental.pallas import tpu_sc as plsc

_REGS = 1e-05
_NC, _NS = 2, 16
_NW = _NC * _NS
_D = 16
_NEG = -1e30


def _iota16():
    return lax.iota(jnp.int32, 16)


def _splat_i32(x):
    return jnp.broadcast_to(jnp.asarray(x, jnp.int32), (16,))


def _make_sc_stats(B, L):
    R = B // _NW
    G = (L + 15) // 16
    LP = G * 16
    LA = 96
    LB = L - LA
    UC = 128
    C = 64
    assert B % _NW == 0 and R % UC == 0 and LB <= 128 and R % C == 0

    mesh = plsc.VectorSubcoreMesh(
        core_axis_name="c", subcore_axis_name="s",
        num_cores=_NC, num_subcores=_NS)

    @functools.partial(
        pl.kernel,
        out_type=jax.ShapeDtypeStruct((6, B), jnp.float32),
        mesh=mesh,
        compiler_params=pltpu.CompilerParams(use_tc_tiling_on_sc=False,
                                             needs_layout_passes=False),
        scratch_types=[
            pltpu.VMEM((R // UC, UC), jnp.int32),
            pltpu.VMEM((R, _D), jnp.float32),
            pltpu.VMEM((C, L), jnp.int32),
            pltpu.VMEM((C, L), jnp.float32),
            pltpu.VMEM((C, LP), jnp.float32),
            pltpu.VMEM((LA, _D), jnp.float32),
            pltpu.VMEM((LB, _D), jnp.float32),
            pltpu.VMEM((LA, _D), jnp.float32),
            pltpu.VMEM((LB, _D), jnp.float32),
            pltpu.VMEM((16 * (R + 8),), jnp.float32),
            pltpu.SemaphoreType.DMA,
            pltpu.SemaphoreType.DMA,
            pltpu.SemaphoreType.DMA,
        ],
    )
    def sc_stats(user_hbm, items_hbm, reward_hbm, gumbel_hbm, uemb_hbm,
                 iemb_hbm, out_hbm, uidx_v, urows_v, idx_v, rwd_v, gmb_v,
                 ea0_v, eb0_v, ea1_v, eb1_v, out_v,
                 semi, sem0, sem1):
        wid = lax.axis_index("s") * _NC + lax.axis_index("c")
        base = wid * R
        iota = _iota16()
        zeros = jnp.zeros((16,), jnp.float32)
        chan_off = iota * _splat_i32(R + 8)

        for k in range(R // UC):
            pltpu.sync_copy(user_hbm.at[pl.ds(base + k * UC, UC)],
                            uidx_v.at[k])
        cps = [pltpu.async_copy(uemb_hbm.at[uidx_v.at[k]],
                                urows_v.at[pl.ds(k * UC, UC)], semi)
               for k in range(R // UC)]
        for c in cps:
            c.wait()

        def issue(c, ea, eb, sem):
            pltpu.async_copy(iemb_hbm.at[idx_v.at[c, pl.ds(0, LA)]], ea, sem)
            pltpu.async_copy(iemb_hbm.at[idx_v.at[c, pl.ds(LA, LB)]], eb,
                             sem)

        def wait_set(ea, eb, sem):
            pltpu.make_async_copy(iemb_hbm.at[pl.ds(0, LA)], ea, sem).wait()
            pltpu.make_async_copy(iemb_hbm.at[pl.ds(0, LB)], eb, sem).wait()

        def compute_row(r, c, ea, eb):
            u = urows_v[r, :]
            dcols = [(iota + _splat_i32(k)) & _splat_i32(15)
                     for k in range(_D)]
            u_b = [jnp.take_along_axis(u, dcols[k], axis=0)
                   for k in range(_D)]

            sq = [zeros] * 4
            se_v = zeros
            best = jnp.full((16,), _NEG, jnp.float32)
            bidx = _splat_i32(0)
            blg = jnp.full((16,), _NEG, jnp.float32)
            c_s = _splat_i32(0) + c
            for j in range(G):
                rows = _splat_i32(j * 16) + iota
                full = (j + 1) * 16 <= L
                in_a = j * 16 < LA
                eref = ea if in_a else eb
                if full:
                    rows_c = rows if in_a else rows - LA
                else:
                    valid = rows < L
                    rows_c = jnp.minimum(rows, L - 1) - LA
                acc = [zeros, zeros]
                for d in range(_D):
                    col = plsc.load_gather(eref, [rows_c, dcols[d]])
                    acc[d & 1] = col * u_b[d] + acc[d & 1]
                    if full:
                        sq[d & 3] = col * col + sq[d & 3]
                    else:
                        sq[d & 3] = jnp.where(valid, col * col + sq[d & 3],
                                              sq[d & 3])
                lg = acc[0] + acc[1]
                if not full:
                    lg = jnp.where(valid, lg, _NEG)
                se_v = se_v + jnp.exp(lg)
                nz = lg + gmb_v[c, pl.ds(j * 16, 16)]
                upd = nz > best
                best = jnp.where(upd, nz, best)
                bidx = jnp.where(upd, _splat_i32(j * 16) + iota, bidx)
                blg = jnp.where(upd, lg, blg)
            se = jnp.sum(se_v)

            mx = jnp.max(best)
            midx = jnp.where(best == mx, bidx, _splat_i32(2 ** 30))
            s_id = jnp.min(midx)
            win = midx == _splat_i32(0) + s_id
            sl = jnp.max(jnp.where(win, blg, _NEG))
            rw_g = plsc.load_gather(rwd_v, [c_s, _splat_i32(0) + s_id])
            rw = jnp.max(rw_g)

            u2 = jnp.sum(u * u)
            i2 = jnp.sum(sq[0] + sq[1] + (sq[2] + sq[3]))

            st = jnp.where(iota == 0, sl, zeros)
            st = jnp.where(iota == 2, se, st)
            st = jnp.where(iota == 3, rw, st)
            st = jnp.where(iota == 4, u2, st)
            st = jnp.where(iota == 5, i2, st)
            plsc.store_scatter(out_v, [chan_off + _splat_i32(0) + r], st)

        def chunk_body(ci, carry):
            cbase = ci * C
            rb = pl.multiple_of(base + cbase, 8)
            c1 = pltpu.async_copy(items_hbm.at[pl.ds(rb, C)], idx_v, semi)
            c2 = pltpu.async_copy(reward_hbm.at[pl.ds(rb, C)], rwd_v, semi)
            c3 = pltpu.async_copy(gumbel_hbm.at[pl.ds(rb, C)],
                                  gmb_v.at[:, pl.ds(0, L)], semi)
            c1.wait()
            c2.wait()
            c3.wait()
            issue(0, ea0_v, eb0_v, sem0)
            issue(1, ea1_v, eb1_v, sem1)

            def pair_body(p, carry2):
                c0 = 2 * p
                wait_set(ea0_v, eb0_v, sem0)
                compute_row(cbase + c0, c0, ea0_v, eb0_v)

                @pl.when(c0 + 2 < C)
                def _():
                    issue(c0 + 2, ea0_v, eb0_v, sem0)

                wait_set(ea1_v, eb1_v, sem1)
                compute_row(cbase + c0 + 1, c0 + 1, ea1_v, eb1_v)

                @pl.when(c0 + 3 < C)
                def _():
                    issue(c0 + 3, ea1_v, eb1_v, sem1)

                return carry2

            lax.fori_loop(0, C // 2, pair_body, 0)
            return carry

        lax.fori_loop(0, R // C, chunk_body, 0)
        for ch in range(6):
            pltpu.sync_copy(out_v.at[pl.ds(ch * (R + 8), R)],
                            out_hbm.at[ch, pl.ds(pl.multiple_of(base, 8), R)])

    return sc_stats


def _tc_reduce(stats, B):
    def body(s_ref, gan_ref, reg_ref):
        s = s_ref[...]
        sl = s[0:1, :]
        se = s[2:3, :]
        rw = s[3:4, :]
        lp = sl - jnp.log(se)
        gan_ref[...] = jnp.reshape(-jnp.mean(lp * rw), (1, 1))
        reg_ref[...] = jnp.reshape(
            _REGS * 0.5 * (jnp.sum(s[4:5, :]) + jnp.sum(s[5:6, :])), (1, 1))

    gan, reg = pl.pallas_call(
        body,
        out_shape=(jax.ShapeDtypeStruct((1, 1), jnp.float32),
                   jax.ShapeDtypeStruct((1, 1), jnp.float32)),
    )(stats)
    return gan[0, 0], reg[0, 0]


def kernel(user, items, reward, user_embedding, item_embedding):
    B, L = items.shape
    gumbel = jax.random.gumbel(jax.random.key(42), (B, L), jnp.float32)
    sc = _make_sc_stats(B, L)
    stats = sc(user, items, reward, gumbel, user_embedding, item_embedding)
    return _tc_reduce(stats, B)
